# baseline (speedup 1.0000x reference)
import jax
import jax.numpy as jnp
from jax.experimental import pallas as pl
from jax.experimental.pallas import tpu as pltpu

N_CHANNELS = 4096
N_BINS = 8192
BLOCK_ROWS = 512


def _body(params_ref, energies_ref, tm_ref, out_ref, flux_ref):
    @pl.when(pl.program_id(0) == 0)
    def _():
        alpha = params_ref[0]
        norm = params_ref[1]
        p = 1.0 - alpha
        e_low = energies_ref[0:1, :]
        e_high = energies_ref[1:2, :]
        flux_ref[...] = norm * (jnp.power(e_high, p) - jnp.power(e_low, p)) / p

    tile = tm_ref[...]
    acc = jnp.sum(tile * flux_ref[...], axis=1, keepdims=True)
    out_ref[...] = jnp.maximum(acc, 1e-6)


def kernel(parameters, transfer_matrix, energies):
    grid = (N_CHANNELS // BLOCK_ROWS,)
    out = pl.pallas_call(
        _body,
        grid=grid,
        in_specs=[
            pl.BlockSpec(memory_space=pltpu.SMEM),
            pl.BlockSpec((2, N_BINS), lambda i: (0, 0)),
            pl.BlockSpec((BLOCK_ROWS, N_BINS), lambda i: (i, 0)),
        ],
        out_specs=pl.BlockSpec((BLOCK_ROWS, 1), lambda i: (i, 0)),
        out_shape=jax.ShapeDtypeStruct((N_CHANNELS, 1), jnp.float32),
        scratch_shapes=[pltpu.VMEM((1, N_BINS), jnp.float32)],
    )(parameters, energies, transfer_matrix)
    return out.reshape(N_CHANNELS)


# speedup vs baseline: 1.0683x; 1.0683x over previous
import jax
import jax.numpy as jnp
from jax.experimental import pallas as pl
from jax.experimental.pallas import tpu as pltpu

N_CHANNELS = 4096
N_BINS = 8192
BLOCK_ROWS = 256


def _body(params_ref, energies_ref, tm_ref, out_ref, flux_ref):
    @pl.when(pl.program_id(0) == 0)
    def _():
        alpha = params_ref[0]
        norm = params_ref[1]
        p = 1.0 - alpha
        e_low = energies_ref[0:1, :]
        e_high = energies_ref[1:2, :]
        flux_ref[...] = norm * (jnp.power(e_high, p) - jnp.power(e_low, p)) / p

    tile = tm_ref[...]
    acc = jnp.sum(tile * flux_ref[...], axis=1, keepdims=True)
    out_ref[...] = jnp.maximum(acc, 1e-6)


def kernel(parameters, transfer_matrix, energies):
    grid = (N_CHANNELS // BLOCK_ROWS,)
    out = pl.pallas_call(
        _body,
        grid=grid,
        in_specs=[
            pl.BlockSpec(memory_space=pltpu.SMEM),
            pl.BlockSpec((2, N_BINS), lambda i: (0, 0)),
            pl.BlockSpec((BLOCK_ROWS, N_BINS), lambda i: (i, 0)),
        ],
        out_specs=pl.BlockSpec((BLOCK_ROWS, 1), lambda i: (i, 0)),
        out_shape=jax.ShapeDtypeStruct((N_CHANNELS, 1), jnp.float32),
        scratch_shapes=[pltpu.VMEM((1, N_BINS), jnp.float32)],
    )(parameters, energies, transfer_matrix)
    return out.reshape(N_CHANNELS)
